# Initial kernel scaffold; baseline (speedup 1.0000x reference)
import jax
import jax.numpy as jnp
from jax.experimental import pallas as pl


def kernel(vertices_0, vertices_1, vertices_2, vertices_3, feats_0, feats_1, feats_2, feats_3, coords_0, coords_1, dn_W1, dn_b1, dn_W2, dn_b2, g_W1, g_b1, g_W2, g_b2, m_W1, m_b1, m_W2, m_b2):
    raise NotImplementedError("write your pallas kernel here")


# speedup vs baseline: 354.3112x; 354.3112x over previous
import functools

import jax
import jax.numpy as jnp
from jax.experimental import pallas as pl
from jax.experimental.pallas import tpu as pltpu

N_COORDS = 1024
N_VERTS = 8192
VBLK = 2048
NB = N_VERTS // VBLK


def _rbf_proj_kernel(coords_ref, verts_ref, feats_ref, w1_ref, b1_ref,
                     w2_ref, b2_ref, out_ref, acc_ref):
    j = pl.program_id(1)

    @pl.when(j == 0)
    def _init():
        acc_ref[...] = jnp.zeros_like(acc_ref)

    c = coords_ref[0]
    v = verts_ref[0]
    f = feats_ref[0]

    h = jnp.maximum(
        jax.lax.dot(f, w1_ref[...], preferred_element_type=jnp.float32)
        + b1_ref[...], 0.0)
    p = jnp.maximum(
        jax.lax.dot(h, w2_ref[...], preferred_element_type=jnp.float32)
        + b2_ref[...], 0.0)

    col = jax.lax.broadcasted_iota(jnp.int32, (VBLK, 128), 1)
    p_aug = jnp.where(col < 64,
                      jnp.pad(p, ((0, 0), (0, 64))),
                      jnp.where(col == 64, 1.0, 0.0))

    cn = jnp.sum(c * c, axis=1)[:, None]
    vn = jnp.sum(v * v, axis=1)[None, :]
    cross = (c[:, 0:1] * v[:, 0][None, :]
             + c[:, 1:2] * v[:, 1][None, :]
             + c[:, 2:3] * v[:, 2][None, :])
    d2 = jnp.maximum(cn + vn - 2.0 * cross, 1e-12)
    w = jnp.exp(-jnp.sqrt(d2) * (1.0 / 2.5))

    acc_ref[...] += jax.lax.dot(w, p_aug, preferred_element_type=jnp.float32)

    @pl.when(j == NB - 1)
    def _flush():
        acc = acc_ref[...]
        ws = acc[:, 64:65] + 0.01
        res = acc / ws
        ocol = jax.lax.broadcasted_iota(jnp.int32, (N_COORDS, 128), 1)
        out_ref[0] = jnp.where(ocol == 64, ws, res)


def _gcn_head_kernel(coords_ref, proj_ref, gw1_ref, gb1_ref, gw2_ref,
                     gb2_ref, mw1_ref, mb1_ref, mw2_ref, mb2_ref,
                     out_ref, acc_ref):
    g = pl.program_id(0)
    c = coords_ref[0]
    x = proj_ref[0]

    cn = jnp.sum(c * c, axis=1)
    cross = (c[:, 0:1] * c[:, 0][None, :]
             + c[:, 1:2] * c[:, 1][None, :]
             + c[:, 2:3] * c[:, 2][None, :])
    d2 = jnp.maximum(cn[:, None] + cn[None, :] - 2.0 * cross, 1e-12)
    a = jnp.exp(-jnp.sqrt(d2) * 0.25)
    ri = jax.lax.broadcasted_iota(jnp.int32, (N_COORDS, N_COORDS), 0)
    ci = jax.lax.broadcasted_iota(jnp.int32, (N_COORDS, N_COORDS), 1)
    a = jnp.where(ri == ci, 1.0, a)

    deg = jnp.sum(a, axis=1)
    dinv = jax.lax.rsqrt(deg)
    an = a * dinv[:, None] * dinv[None, :]

    xw = jax.lax.dot(x, gw1_ref[...], preferred_element_type=jnp.float32)
    h = jnp.maximum(
        jax.lax.dot(an, xw, preferred_element_type=jnp.float32)
        + gb1_ref[...], 0.0)
    hw = jax.lax.dot(h, gw2_ref[...], preferred_element_type=jnp.float32)
    nodes = (jax.lax.dot(an, hw, preferred_element_type=jnp.float32)
             + gb2_ref[...])

    colsum = jnp.sum(nodes, axis=0)[None, :]

    @pl.when(g == 0)
    def _init():
        acc_ref[...] = colsum

    @pl.when(g == 1)
    def _final():
        m = (acc_ref[...] + colsum) * (1.0 / (2.0 * N_COORDS))
        y = jnp.maximum(
            jax.lax.dot(m, mw1_ref[...], preferred_element_type=jnp.float32)
            + mb1_ref[...], 0.0)
        z = jax.lax.dot(y, mw2_ref[...], preferred_element_type=jnp.float32)
        out_ref[...] = jax.nn.sigmoid(z + mb2_ref[...])


@jax.jit
def kernel(vertices_0, vertices_1, vertices_2, vertices_3,
           feats_0, feats_1, feats_2, feats_3, coords_0, coords_1,
           dn_W1, dn_b1, dn_W2, dn_b2, g_W1, g_b1, g_W2, g_b2,
           m_W1, m_b1, m_W2, m_b2):
    verts = jnp.stack([vertices_0, vertices_1, vertices_2, vertices_3])
    feats = jnp.stack([feats_0, feats_1, feats_2, feats_3])
    coords = jnp.stack([coords_0, coords_1])

    proj = pl.pallas_call(
        _rbf_proj_kernel,
        grid=(4, NB),
        in_specs=[
            pl.BlockSpec((1, N_COORDS, 3), lambda s, j: (s // 2, 0, 0)),
            pl.BlockSpec((1, VBLK, 3), lambda s, j: (s, j, 0)),
            pl.BlockSpec((1, VBLK, 5), lambda s, j: (s, j, 0)),
            pl.BlockSpec((5, 10), lambda s, j: (0, 0)),
            pl.BlockSpec((1, 10), lambda s, j: (0, 0)),
            pl.BlockSpec((10, 64), lambda s, j: (0, 0)),
            pl.BlockSpec((1, 64), lambda s, j: (0, 0)),
        ],
        out_specs=pl.BlockSpec((1, N_COORDS, 128), lambda s, j: (s, 0, 0)),
        out_shape=jax.ShapeDtypeStruct((4, N_COORDS, 128), jnp.float32),
        scratch_shapes=[pltpu.VMEM((N_COORDS, 128), jnp.float32)],
        compiler_params=pltpu.CompilerParams(
            dimension_semantics=("arbitrary", "arbitrary")),
    )(coords, verts, feats, dn_W1, dn_b1.reshape(1, 10),
      dn_W2, dn_b2.reshape(1, 64))

    p = proj[:, :, :65]
    projs = jnp.stack([jnp.concatenate([p[0], p[1]], axis=1),
                       jnp.concatenate([p[2], p[3]], axis=1)])

    res = pl.pallas_call(
        _gcn_head_kernel,
        grid=(2,),
        in_specs=[
            pl.BlockSpec((1, N_COORDS, 3), lambda g: (g, 0, 0)),
            pl.BlockSpec((1, N_COORDS, 130), lambda g: (g, 0, 0)),
            pl.BlockSpec((130, 64), lambda g: (0, 0)),
            pl.BlockSpec((1, 64), lambda g: (0, 0)),
            pl.BlockSpec((64, 64), lambda g: (0, 0)),
            pl.BlockSpec((1, 64), lambda g: (0, 0)),
            pl.BlockSpec((64, 128), lambda g: (0, 0)),
            pl.BlockSpec((1, 128), lambda g: (0, 0)),
            pl.BlockSpec((128, 1), lambda g: (0, 0)),
            pl.BlockSpec((1, 1), lambda g: (0, 0)),
        ],
        out_specs=pl.BlockSpec((1, 1), lambda g: (0, 0)),
        out_shape=jax.ShapeDtypeStruct((1, 1), jnp.float32),
        scratch_shapes=[pltpu.VMEM((1, 64), jnp.float32)],
        compiler_params=pltpu.CompilerParams(
            dimension_semantics=("arbitrary",)),
    )(coords, projs, g_W1, g_b1.reshape(1, 64), g_W2, g_b2.reshape(1, 64),
      m_W1, m_b1.reshape(1, 128), m_W2, m_b2.reshape(1, 1))

    return res[0]
